# baseline (speedup 1.0000x reference)
import functools

import jax
import jax.numpy as jnp
from jax.experimental import pallas as pl
from jax.experimental.pallas import tpu as pltpu

EDGE_BLK = 16384
NODE_BLK = 8192


def _edge_mlp_body(xs0, xs1, s, e, w1a, b1a, w1b, b1b, out):
    h1 = (
        xs0[:][:, None] * w1a[0, :][None, :]
        + xs1[:][:, None] * w1a[1, :][None, :]
        + s[:][:, None] * w1a[2, :][None, :]
        + e[:][:, None] * w1a[3, :][None, :]
        + b1a[:][None, :]
    )
    h1 = jnp.maximum(h1, 0.0)
    m = jnp.dot(h1, w1b[:], preferred_element_type=jnp.float32) + b1b[:][None, :]
    out[:] = jnp.maximum(m, 0.0)


def _edge_mlp(xs0, xs1, s, e, W1a, b1a, W1b, b1b):
    E = xs0.shape[0]
    grid = (E // EDGE_BLK,)
    vec_spec = pl.BlockSpec((EDGE_BLK,), lambda i: (i,))
    full = lambda shp: pl.BlockSpec(shp, lambda i: tuple(0 for _ in shp))
    return pl.pallas_call(
        _edge_mlp_body,
        grid=grid,
        in_specs=[
            vec_spec, vec_spec, vec_spec, vec_spec,
            full((4, 32)), full((32,)), full((32, 32)), full((32,)),
        ],
        out_specs=pl.BlockSpec((EDGE_BLK, 32), lambda i: (i, 0)),
        out_shape=jax.ShapeDtypeStruct((E, 32), jnp.float32),
    )(xs0, xs1, s, e, W1a, b1a, W1b, b1b)


def _node_mlp_body(x0, x1, x2, aggr, w2x, w2a, b2a, w2b, b2b, out):
    h = (
        x0[:][:, None] * w2x[0, :][None, :]
        + x1[:][:, None] * w2x[1, :][None, :]
        + x2[:][:, None] * w2x[2, :][None, :]
        + jnp.dot(aggr[:], w2a[:], preferred_element_type=jnp.float32)
        + b2a[:][None, :]
    )
    h = jnp.maximum(h, 0.0)
    z = jnp.sum(h * w2b[:][None, :], axis=1) + b2b[0]
    out[:] = jax.nn.sigmoid(z)


def _node_mlp(x0, x1, x2, aggr, W2a, b2a, W2b, b2b):
    N = x0.shape[0]
    grid = (pl.cdiv(N, NODE_BLK),)
    vec_spec = pl.BlockSpec((NODE_BLK,), lambda i: (i,))
    full = lambda shp: pl.BlockSpec(shp, lambda i: tuple(0 for _ in shp))
    w2x = W2a[:3]
    w2a = W2a[3:]
    w2b = W2b[:, 0]
    return pl.pallas_call(
        _node_mlp_body,
        grid=grid,
        in_specs=[
            vec_spec, vec_spec, vec_spec,
            pl.BlockSpec((NODE_BLK, 32), lambda i: (i, 0)),
            full((3, 16)), full((32, 16)), full((16,)), full((16,)), full((1,)),
        ],
        out_specs=vec_spec,
        out_shape=jax.ShapeDtypeStruct((N,), jnp.float32),
    )(x0, x1, x2, aggr, w2x, w2a, b2a, W2b[:, 0], b2b)


def kernel(x, edge_index, edge_attr, W1a, b1a, W1b, b1b, W2a, b2a, W2b, b2b):
    N = x.shape[0]
    E = edge_index.shape[1]
    src = edge_index[0]
    dst = edge_index[1]
    e = edge_attr[:, 0]

    x0 = x[:, 0]
    x1 = x[:, 1]
    x2 = x[:, 2]

    E_pad = pl.cdiv(E, EDGE_BLK) * EDGE_BLK
    srcp = jnp.pad(src, (0, E_pad - E))
    ep = jnp.pad(e, (0, E_pad - E))
    dstp = jnp.pad(dst, (0, E_pad - E), constant_values=N)

    xs0 = jnp.take(x0, srcp, axis=0)
    xs1 = jnp.take(x1, srcp, axis=0)

    for _ in range(3):
        s = jnp.take(x2, srcp, axis=0)
        m = _edge_mlp(xs0, xs1, s, ep, W1a, b1a, W1b, b1b)
        aggr = jax.ops.segment_max(m, dstp, num_segments=N + 1)[:N]
        aggr = jnp.maximum(aggr, 0.0)
        x2 = _node_mlp(x0, x1, x2, aggr, W2a, b2a, W2b, b2b)

    return jnp.concatenate([x[:, :2], x2[:, None]], axis=1)


# speedup vs baseline: 6.2975x; 6.2975x over previous
import functools

import jax
import jax.numpy as jnp
from jax import lax
from jax.experimental import pallas as pl
from jax.experimental.pallas import tpu as pltpu
from jax.experimental.pallas import tpu_sc as plsc

EDGE_BLK = 16384
NODE_BLK = 8192

SC_CORES = 2
SC_SUBCORES = 16
SC_TILES = SC_CORES * SC_SUBCORES

_SC_MESH = dict(core_axis_name="c", subcore_axis_name="s")


def _tile_id():
    return lax.axis_index("s") * SC_CORES + lax.axis_index("c")


def _sc_gather(table, idx, chunk=2048):
    E = idx.shape[0]
    assert E % (SC_TILES * chunk) == 0, (E, chunk)
    per_tile = E // SC_TILES
    n_chunks = per_tile // chunk

    @functools.partial(
        pl.kernel,
        out_type=jax.ShapeDtypeStruct((E,), jnp.float32),
        mesh=plsc.VectorSubcoreMesh(**_SC_MESH),
        scratch_types=[
            pltpu.VMEM((chunk,), jnp.int32),
            pltpu.VMEM((chunk,), jnp.float32),
            pltpu.SemaphoreType.DMA,
        ],
    )
    def run(table_hbm, idx_hbm, out_hbm, idx_v, val_v, sem):
        base = _tile_id() * per_tile

        def body(k, carry):
            off = base + k * chunk
            pltpu.sync_copy(idx_hbm.at[pl.ds(off, chunk)], idx_v)
            pltpu.async_copy(table_hbm.at[idx_v], val_v, sem).wait()
            pltpu.sync_copy(val_v, out_hbm.at[pl.ds(off, chunk)])
            return carry

        lax.fori_loop(0, n_chunks, body, 0)

    return run(table, idx)


def _edge_mlp_body(xs0, xs1, s, e, w1a, b1a, w1b, b1b, out):
    h1 = (
        xs0[:][:, None] * w1a[0, :][None, :]
        + xs1[:][:, None] * w1a[1, :][None, :]
        + s[:][:, None] * w1a[2, :][None, :]
        + e[:][:, None] * w1a[3, :][None, :]
        + b1a[:][None, :]
    )
    h1 = jnp.maximum(h1, 0.0)
    m = jnp.dot(h1, w1b[:], preferred_element_type=jnp.float32) + b1b[:][None, :]
    out[:] = jnp.maximum(m, 0.0)


def _edge_mlp(xs0, xs1, s, e, W1a, b1a, W1b, b1b):
    E = xs0.shape[0]
    grid = (E // EDGE_BLK,)
    vec_spec = pl.BlockSpec((EDGE_BLK,), lambda i: (i,))
    full = lambda shp: pl.BlockSpec(shp, lambda i: tuple(0 for _ in shp))
    return pl.pallas_call(
        _edge_mlp_body,
        grid=grid,
        in_specs=[
            vec_spec, vec_spec, vec_spec, vec_spec,
            full((4, 32)), full((32,)), full((32, 32)), full((32,)),
        ],
        out_specs=pl.BlockSpec((EDGE_BLK, 32), lambda i: (i, 0)),
        out_shape=jax.ShapeDtypeStruct((E, 32), jnp.float32),
    )(xs0, xs1, s, e, W1a, b1a, W1b, b1b)


def _node_mlp_body(x0, x1, x2, aggr, w2x, w2a, b2a, w2b, b2b, out):
    h = (
        x0[:][:, None] * w2x[0, :][None, :]
        + x1[:][:, None] * w2x[1, :][None, :]
        + x2[:][:, None] * w2x[2, :][None, :]
        + jnp.dot(aggr[:], w2a[:], preferred_element_type=jnp.float32)
        + b2a[:][None, :]
    )
    h = jnp.maximum(h, 0.0)
    z = jnp.sum(h * w2b[:][None, :], axis=1) + b2b[0]
    out[:] = jax.nn.sigmoid(z)


def _node_mlp(x0, x1, x2, aggr, W2a, b2a, W2b, b2b):
    N = x0.shape[0]
    grid = (pl.cdiv(N, NODE_BLK),)
    vec_spec = pl.BlockSpec((NODE_BLK,), lambda i: (i,))
    full = lambda shp: pl.BlockSpec(shp, lambda i: tuple(0 for _ in shp))
    w2x = W2a[:3]
    w2a = W2a[3:]
    w2b = W2b[:, 0]
    return pl.pallas_call(
        _node_mlp_body,
        grid=grid,
        in_specs=[
            vec_spec, vec_spec, vec_spec,
            pl.BlockSpec((NODE_BLK, 32), lambda i: (i, 0)),
            full((3, 16)), full((32, 16)), full((16,)), full((16,)), full((1,)),
        ],
        out_specs=vec_spec,
        out_shape=jax.ShapeDtypeStruct((N,), jnp.float32),
    )(x0, x1, x2, aggr, w2x, w2a, b2a, W2b[:, 0], b2b)


def kernel(x, edge_index, edge_attr, W1a, b1a, W1b, b1b, W2a, b2a, W2b, b2b):
    N = x.shape[0]
    E = edge_index.shape[1]
    src = edge_index[0]
    dst = edge_index[1]
    e = edge_attr[:, 0]

    x0 = x[:, 0]
    x1 = x[:, 1]
    x2 = x[:, 2]

    E_pad = pl.cdiv(E, EDGE_BLK) * EDGE_BLK
    srcp = jnp.pad(src, (0, E_pad - E))
    ep = jnp.pad(e, (0, E_pad - E))
    dstp = jnp.pad(dst, (0, E_pad - E), constant_values=N)

    xs0 = _sc_gather(x0, srcp)
    xs1 = _sc_gather(x1, srcp)

    for _ in range(3):
        s = _sc_gather(x2, srcp)
        m = _edge_mlp(xs0, xs1, s, ep, W1a, b1a, W1b, b1b)
        aggr = jax.ops.segment_max(m, dstp, num_segments=N + 1)[:N]
        aggr = jnp.maximum(aggr, 0.0)
        x2 = _node_mlp(x0, x1, x2, aggr, W2a, b2a, W2b, b2b)

    return jnp.concatenate([x[:, :2], x2[:, None]], axis=1)
